# baseline (speedup 1.0000x reference)
import functools

import jax
import jax.numpy as jnp
from jax import lax
from jax.experimental import pallas as pl
from jax.experimental.pallas import tpu as pltpu
from jax.experimental.pallas import tpu_sc as plsc

_BATCH = 16384
_DIM = 64
_LANES = 16


def _mf_kernel(user_hbm, item_hbm, utab_hbm, itab_hbm, out_hbm,
               uidx_v, iidx_v, urows_v, irows_v, out_v, usem, isem):
    nc = 2
    wid = lax.axis_index("s") * nc + lax.axis_index("c")
    b_per_w = _BATCH // 32
    base = wid * b_per_w

    pltpu.sync_copy(user_hbm.at[pl.ds(base, b_per_w)], uidx_v)
    pltpu.sync_copy(item_hbm.at[pl.ds(base, b_per_w)], iidx_v)

    ucopy = pltpu.async_copy(utab_hbm.at[uidx_v], urows_v, usem)
    icopy = pltpu.async_copy(itab_hbm.at[iidx_v], irows_v, isem)
    ucopy.wait()
    icopy.wait()

    iota = lax.iota(jnp.int32, _LANES)

    def group_body(g, _):
        rows = g * _LANES + iota
        acc = jnp.zeros((_LANES,), jnp.float32)
        for d in range(_DIM):
            cols = jnp.full((_LANES,), d, jnp.int32)
            u = plsc.load_gather(urows_v, [rows, cols])
            v = plsc.load_gather(irows_v, [rows, cols])
            acc = acc + u * v
        out_v[pl.ds(g * _LANES, _LANES)] = acc
        return 0

    lax.fori_loop(0, b_per_w // _LANES, group_body, 0)

    pltpu.sync_copy(out_v, out_hbm.at[pl.ds(base, b_per_w)])


@functools.partial(jax.jit, static_argnames=())
def kernel(user, item, user_table, item_table):
    b_per_w = _BATCH // 32
    mesh = plsc.VectorSubcoreMesh(core_axis_name="c", subcore_axis_name="s")
    run = pl.kernel(
        _mf_kernel,
        mesh=mesh,
        compiler_params=pltpu.CompilerParams(
            use_tc_tiling_on_sc=False,
            needs_layout_passes=False,
        ),
        out_type=jax.ShapeDtypeStruct((_BATCH,), jnp.float32),
        scratch_types=[
            pltpu.VMEM((b_per_w,), jnp.int32),
            pltpu.VMEM((b_per_w,), jnp.int32),
            pltpu.VMEM((b_per_w, _DIM), jnp.float32),
            pltpu.VMEM((b_per_w, _DIM), jnp.float32),
            pltpu.VMEM((b_per_w,), jnp.float32),
            pltpu.SemaphoreType.DMA,
            pltpu.SemaphoreType.DMA,
        ],
    )
    return run(user, item, user_table, item_table)


# speedup vs baseline: 1.8958x; 1.8958x over previous
import functools

import jax
import jax.numpy as jnp
from jax import lax
from jax.experimental import pallas as pl
from jax.experimental.pallas import tpu as pltpu
from jax.experimental.pallas import tpu_sc as plsc

_BATCH = 16384
_DIM = 64
_LANES = 16
_NW = 32
_NCOLS = (1000000 + 127) // 128
_CPW = (_NCOLS + _NW - 1) // _NW
_PAIR_CAP = _BATCH + 16
_NGRP = (_BATCH + 127) // 128
_ICHUNK = 2048


def _scan_sort(idx_hbm, idx_chunk_v, pairs_v, sorted_v, bidx_v,
               hist_s, cum_s, lo, nc):
    iota = lax.iota(jnp.int32, _LANES)

    def init_b(r, _):
        for k in range(128 // _LANES):
            bidx_v[r, pl.ds(k * _LANES, _LANES)] = jnp.full(
                (_LANES,), _BATCH, jnp.int32)
        return 0
    lax.fori_loop(0, _NGRP + 1, init_b, 0)

    def init_h(c, _):
        hist_s[c] = 0
        return 0
    lax.fori_loop(0, 256, init_h, 0)

    def chunk_body(ch, cnt):
        pltpu.sync_copy(idx_hbm.at[pl.ds(ch * _ICHUNK, _ICHUNK)], idx_chunk_v)

        def scan_body(g, cnt):
            u = idx_chunk_v[pl.ds(g * _LANES, _LANES)]
            col = u >> 7
            rel = col - lo
            m = (rel >= 0) & (rel < nc)
            b_vec = ch * _ICHUNK + g * _LANES + iota
            val = ((b_vec << 7) | (u & 127)) << 8 | rel
            plsc.store_compressed(pairs_v.at[pl.ds(cnt, _LANES)], val, mask=m)
            pc = plsc.all_reduce_population_count(m)
            return cnt + pc[0]

        return lax.fori_loop(0, _ICHUNK // _LANES, scan_body, cnt)

    cnt = lax.fori_loop(0, _BATCH // _ICHUNK, chunk_body, 0)

    def hist_body(k, _):
        v = pairs_v[pl.ds(k * _LANES, _LANES)]
        for j in range(_LANES):
            @pl.when(k * _LANES + j < cnt)
            def _():
                c = v[j] & 255
                hist_s[c] = hist_s[c] + 1
        return 0
    lax.fori_loop(0, (cnt + _LANES - 1) // _LANES, hist_body, 0)

    def pfx_body(c, acc):
        cum_s[c] = acc
        n = hist_s[c]
        hist_s[c] = acc
        return acc + n
    total = lax.fori_loop(0, nc, pfx_body, 0)
    cum_s[nc] = total

    def place_body(k, _):
        v = pairs_v[pl.ds(k * _LANES, _LANES)]
        pos = jnp.zeros((_LANES,), jnp.int32)
        for j in range(_LANES):
            c = v[j] & 255
            p = hist_s[c]
            pos = jnp.where(iota == j, p, pos)
            @pl.when(k * _LANES + j < cnt)
            def _():
                hist_s[c] = p + 1
        valid = (k * _LANES + iota) < cnt
        data = v >> 8
        plsc.store_scatter(sorted_v, [pos], data, mask=valid)
        plsc.store_scatter(bidx_v, [pos >> 7, pos & 127], data >> 7,
                           mask=valid)
        return 0
    lax.fori_loop(0, (cnt + _LANES - 1) // _LANES, place_body, 0)
    return cnt


def _select_scalar(vec, j):
    iota = lax.iota(jnp.int32, _LANES)
    return jnp.sum(jnp.where(iota == j, vec, 0))


def _extract_row(colbuf2d, lane, dst3, g, r):
    iota = lax.iota(jnp.int32, _LANES)
    lane_vec = jnp.full((_LANES,), lane, jnp.int32)
    for k in range(_DIM // _LANES):
        dvec = k * _LANES + iota
        val = plsc.load_gather(colbuf2d, [dvec, lane_vec])
        dst3[g, r, pl.ds(k * _LANES, _LANES)] = val


def _col_dma(tabT_hbm, lo, cc, colbuf_v, slot, sem):
    return pltpu.async_copy(
        tabT_hbm.at[:, pl.ds((lo + cc) * 128, 128)], colbuf_v.at[slot], sem)


def _users_kernel(user_hbm, utabT_hbm, rows_u_hbm,
                  idx_chunk_v, pairs_v, sorted_v, bidx_v, colbuf_v,
                  rows_seq_v, hist_s, cum_s, csem, wsem):
    wid = lax.axis_index("s") * 2 + lax.axis_index("c")
    lo = wid * _CPW
    nc = jnp.minimum(lo + _CPW, _NCOLS) - lo

    cnt = _scan_sort(user_hbm, idx_chunk_v, pairs_v, sorted_v, bidx_v,
                     hist_s, cum_s, lo, nc)

    def nonempty(cc):
        return cum_s[cc] < cum_s[cc + 1]

    for p in range(2):
        @pl.when((p < nc) & nonempty(p))
        def _():
            _col_dma(utabT_hbm, lo, p, colbuf_v, p, csem)

    def col_body(cc, _):
        begin = cum_s[cc]
        end = cum_s[cc + 1]

        @pl.when(begin < end)
        def _process():
            pltpu.make_async_copy(
                utabT_hbm.at[:, pl.ds(lo * 128, 128)],
                colbuf_v.at[cc & 1], csem).wait()

            def pair_body(i, _):
                v = sorted_v[pl.ds((i >> 4) << 4, _LANES)]
                lb = _select_scalar(v, i & 15)
                lane = lb & 127
                g = (i >> 7) & 1
                r = i & 127
                _extract_row(colbuf_v.at[cc & 1], lane, rows_seq_v, g, r)

                @pl.when((i & 127) == 127)
                def _flush():
                    grp = i >> 7
                    @pl.when(grp >= 1)
                    def _():
                        pltpu.make_async_copy(
                            rows_seq_v.at[0],
                            rows_u_hbm.at[bidx_v.at[0]], wsem).wait()
                    pltpu.async_copy(
                        rows_seq_v.at[g],
                        rows_u_hbm.at[bidx_v.at[grp]], wsem)
                return 0

            lax.fori_loop(begin, end, pair_body, 0)

        @pl.when((cc + 2 < nc) & nonempty(cc + 2))
        def _prefetch():
            _col_dma(utabT_hbm, lo, cc + 2, colbuf_v, cc & 1, csem)
        return 0

    lax.fori_loop(0, nc, col_body, 0)

    last_grp = cnt >> 7
    @pl.when((cnt & 127) != 0)
    def _tail():
        pltpu.async_copy(
            rows_seq_v.at[last_grp & 1],
            rows_u_hbm.at[bidx_v.at[last_grp]], wsem)
    ngrp_fired = (cnt + 127) >> 7
    @pl.when(ngrp_fired >= 1)
    def _d1():
        pltpu.make_async_copy(
            rows_seq_v.at[0], rows_u_hbm.at[bidx_v.at[0]], wsem).wait()
    @pl.when(ngrp_fired >= 2)
    def _d2():
        pltpu.make_async_copy(
            rows_seq_v.at[0], rows_u_hbm.at[bidx_v.at[0]], wsem).wait()


def _items_kernel(item_hbm, itabT_hbm, rows_u_hbm, out_pad_hbm,
                  idx_chunk_v, pairs_v, sorted_v, bidx_v, colbuf_v,
                  rows_seq_v, urows_v, hist_s, cum_s, csem, gsem, wsem):
    wid = lax.axis_index("s") * 2 + lax.axis_index("c")
    lo = wid * _CPW
    nc = jnp.minimum(lo + _CPW, _NCOLS) - lo
    iota = lax.iota(jnp.int32, _LANES)

    cnt = _scan_sort(item_hbm, idx_chunk_v, pairs_v, sorted_v, bidx_v,
                     hist_s, cum_s, lo, nc)

    def nonempty(cc):
        return cum_s[cc] < cum_s[cc + 1]

    def dot_group(grp, g):
        pltpu.async_copy(
            rows_u_hbm.at[bidx_v.at[grp]], urows_v, gsem).wait()
        for sub in range(128 // _LANES):
            rvec = sub * _LANES + iota
            acc = jnp.zeros((_LANES,), jnp.float32)
            for d in range(_DIM):
                dvec = jnp.full((_LANES,), d, jnp.int32)
                uu = plsc.load_gather(urows_v, [rvec, dvec])
                vv = plsc.load_gather(rows_seq_v.at[g], [rvec, dvec])
                acc = acc + uu * vv
            plsc.store_scatter(urows_v, [rvec, jnp.zeros((_LANES,), jnp.int32)],
                               acc)
        pltpu.async_copy(urows_v, out_pad_hbm.at[bidx_v.at[grp]], wsem).wait()

    for p in range(2):
        @pl.when((p < nc) & nonempty(p))
        def _():
            _col_dma(itabT_hbm, lo, p, colbuf_v, p, csem)

    def col_body(cc, _):
        begin = cum_s[cc]
        end = cum_s[cc + 1]

        @pl.when(begin < end)
        def _process():
            pltpu.make_async_copy(
                itabT_hbm.at[:, pl.ds(lo * 128, 128)],
                colbuf_v.at[cc & 1], csem).wait()

            def pair_body(i, _):
                v = sorted_v[pl.ds((i >> 4) << 4, _LANES)]
                lb = _select_scalar(v, i & 15)
                lane = lb & 127
                g = (i >> 7) & 1
                r = i & 127
                _extract_row(colbuf_v.at[cc & 1], lane, rows_seq_v, g, r)

                @pl.when((i & 127) == 127)
                def _flush():
                    dot_group(i >> 7, g)
                return 0

            lax.fori_loop(begin, end, pair_body, 0)

        @pl.when((cc + 2 < nc) & nonempty(cc + 2))
        def _prefetch():
            _col_dma(itabT_hbm, lo, cc + 2, colbuf_v, cc & 1, csem)
        return 0

    lax.fori_loop(0, nc, col_body, 0)

    last_grp = cnt >> 7
    @pl.when((cnt & 127) != 0)
    def _tail():
        dot_group(last_grp, last_grp & 1)


_COMPILER_PARAMS = pltpu.CompilerParams(
    use_tc_tiling_on_sc=True,
    needs_layout_passes=False,
)


@functools.partial(jax.jit, static_argnames=())
def kernel(user, item, user_table, item_table):
    mesh = plsc.VectorSubcoreMesh(core_axis_name="c", subcore_axis_name="s")

    run_users = pl.kernel(
        _users_kernel,
        mesh=mesh,
        compiler_params=_COMPILER_PARAMS,
        out_type=jax.ShapeDtypeStruct((_BATCH + 1, 128), jnp.float32),
        scratch_types=[
            pltpu.VMEM((_ICHUNK,), jnp.int32),
            pltpu.VMEM((_PAIR_CAP,), jnp.int32),
            pltpu.VMEM((_PAIR_CAP,), jnp.int32),
            pltpu.VMEM((_NGRP + 1, 128), jnp.int32),
            pltpu.VMEM((2, _DIM, 128), jnp.float32),
            pltpu.VMEM((2, 128, 128), jnp.float32),
            pltpu.SMEM((256,), jnp.int32),
            pltpu.SMEM((256,), jnp.int32),
            pltpu.SemaphoreType.DMA,
            pltpu.SemaphoreType.DMA,
        ],
    )
    rows_u = run_users(user, user_table.T)

    run_items = pl.kernel(
        _items_kernel,
        mesh=mesh,
        compiler_params=_COMPILER_PARAMS,
        out_type=jax.ShapeDtypeStruct((_BATCH + 1, 128), jnp.float32),
        scratch_types=[
            pltpu.VMEM((_ICHUNK,), jnp.int32),
            pltpu.VMEM((_PAIR_CAP,), jnp.int32),
            pltpu.VMEM((_PAIR_CAP,), jnp.int32),
            pltpu.VMEM((_NGRP + 1, 128), jnp.int32),
            pltpu.VMEM((2, _DIM, 128), jnp.float32),
            pltpu.VMEM((2, 128, 128), jnp.float32),
            pltpu.VMEM((128, 128), jnp.float32),
            pltpu.SMEM((256,), jnp.int32),
            pltpu.SMEM((256,), jnp.int32),
            pltpu.SemaphoreType.DMA,
            pltpu.SemaphoreType.DMA,
            pltpu.SemaphoreType.DMA,
        ],
    )
    out_pad = run_items(item, item_table.T, rows_u)
    return out_pad[:_BATCH, 0]


# speedup vs baseline: 2.3646x; 1.2473x over previous
import functools

import jax
import jax.numpy as jnp
from jax import lax
from jax.experimental import pallas as pl
from jax.experimental.pallas import tpu as pltpu
from jax.experimental.pallas import tpu_sc as plsc

_BATCH = 16384
_DIM = 64
_LANES = 16
_NW = 32
_NCOLS = (1000000 + 127) // 128
_CPW = (_NCOLS + _NW - 1) // _NW
_PAIR_CAP = _BATCH + 16
_NGRP = (_BATCH + 127) // 128
_ICHUNK = 2048


def _scan_sort(idx_hbm, idx_chunk_v, pairs_v, sorted_v, bidx_v,
               hist_s, cum_s, lo, nc):
    iota = lax.iota(jnp.int32, _LANES)

    def init_b(r, _):
        for k in range(128 // _LANES):
            bidx_v[r, pl.ds(k * _LANES, _LANES)] = jnp.full(
                (_LANES,), _BATCH, jnp.int32)
        return 0
    lax.fori_loop(0, _NGRP + 1, init_b, 0)

    def init_h(c, _):
        hist_s[c] = 0
        return 0
    lax.fori_loop(0, 256, init_h, 0)

    def chunk_body(ch, cnt):
        pltpu.sync_copy(idx_hbm.at[pl.ds(ch * _ICHUNK, _ICHUNK)], idx_chunk_v)

        def scan_body(g, cnt):
            u = idx_chunk_v[pl.ds(g * _LANES, _LANES)]
            col = u >> 7
            rel = col - lo
            m = (rel >= 0) & (rel < nc)
            b_vec = ch * _ICHUNK + g * _LANES + iota
            val = ((b_vec << 7) | (u & 127)) << 8 | rel
            plsc.store_compressed(pairs_v.at[pl.ds(cnt, _LANES)], val, mask=m)
            pc = plsc.all_reduce_population_count(m)
            return cnt + pc[0]

        return lax.fori_loop(0, _ICHUNK // _LANES, scan_body, cnt)

    cnt = lax.fori_loop(0, _BATCH // _ICHUNK, chunk_body, 0)

    def hist_body(k, _):
        v = pairs_v[pl.ds(k * _LANES, _LANES)]
        for j in range(_LANES):
            @pl.when(k * _LANES + j < cnt)
            def _():
                c = v[j] & 255
                hist_s[c] = hist_s[c] + 1
        return 0
    lax.fori_loop(0, (cnt + _LANES - 1) // _LANES, hist_body, 0)

    def pfx_body(c, acc):
        cum_s[c] = acc
        n = hist_s[c]
        hist_s[c] = acc
        return acc + n
    total = lax.fori_loop(0, nc, pfx_body, 0)
    cum_s[nc] = total

    def place_body(k, _):
        v = pairs_v[pl.ds(k * _LANES, _LANES)]
        pos = jnp.zeros((_LANES,), jnp.int32)
        for j in range(_LANES):
            c = v[j] & 255
            p = hist_s[c]
            pos = jnp.where(iota == j, p, pos)
            @pl.when(k * _LANES + j < cnt)
            def _():
                hist_s[c] = p + 1
        valid = (k * _LANES + iota) < cnt
        data = v >> 8
        plsc.store_scatter(sorted_v, [pos], data, mask=valid)
        plsc.store_scatter(bidx_v, [pos >> 7, pos & 127], data >> 7,
                           mask=valid)
        return 0
    lax.fori_loop(0, (cnt + _LANES - 1) // _LANES, place_body, 0)
    return cnt


def _select_scalar(vec, j):
    iota = lax.iota(jnp.int32, _LANES)
    return jnp.sum(jnp.where(iota == j, vec, 0))


def _extract_row(colbuf2d, lane, dst2, r):
    iota = lax.iota(jnp.int32, _LANES)
    lane_vec = jnp.full((_LANES,), lane, jnp.int32)
    for k in range(_DIM // _LANES):
        dvec = k * _LANES + iota
        val = plsc.load_gather(colbuf2d, [dvec, lane_vec])
        dst2[r, pl.ds(k * _LANES, _LANES)] = val


def _col_dma(tabT_hbm, lo, cc, colbuf_v, slot, sem):
    return pltpu.async_copy(
        tabT_hbm.at[:, pl.ds((lo + cc) * 128, 128)], colbuf_v.at[slot], sem)


def _users_kernel(user_hbm, utabT_hbm, rows_u_hbm,
                  idx_chunk_v, pairs_v, sorted_v, bidx_v, colbuf_v,
                  rows_seq_v, hist_s, cum_s, csem, wsem):
    wid = lax.axis_index("s") * 2 + lax.axis_index("c")
    lo = wid * _CPW
    nc = jnp.minimum(lo + _CPW, _NCOLS) - lo

    cnt = _scan_sort(user_hbm, idx_chunk_v, pairs_v, sorted_v, bidx_v,
                     hist_s, cum_s, lo, nc)

    def nonempty(cc):
        return cum_s[cc] < cum_s[cc + 1]

    for p in range(4):
        @pl.when((p < nc) & nonempty(p))
        def _():
            _col_dma(utabT_hbm, lo, p, colbuf_v, p, csem)

    def col_body(cc, _):
        begin = cum_s[cc]
        end = cum_s[cc + 1]
        slot = lax.rem(cc, 5)

        @pl.when((cc + 4 < nc) & nonempty(cc + 4))
        def _prefetch():
            _col_dma(utabT_hbm, lo, cc + 4, colbuf_v, lax.rem(cc + 4, 5),
                     csem)

        @pl.when(begin < end)
        def _process():
            pltpu.make_async_copy(
                utabT_hbm.at[:, pl.ds(lo * 128, 128)],
                colbuf_v.at[slot], csem).wait()

            def pair_body(i, _):
                v = sorted_v[pl.ds((i >> 4) << 4, _LANES)]
                lb = _select_scalar(v, i & 15)
                lane = lb & 127
                r = i & 127
                _extract_row(colbuf_v.at[slot], lane, rows_seq_v, r)

                @pl.when((i & 127) == 127)
                def _flush():
                    pltpu.async_copy(
                        rows_seq_v,
                        rows_u_hbm.at[bidx_v.at[i >> 7]], wsem).wait()
                return 0

            lax.fori_loop(begin, end, pair_body, 0)
        return 0

    lax.fori_loop(0, nc, col_body, 0)

    @pl.when((cnt & 127) != 0)
    def _tail():
        pltpu.async_copy(
            rows_seq_v, rows_u_hbm.at[bidx_v.at[cnt >> 7]], wsem).wait()


def _items_kernel(item_hbm, itabT_hbm, rows_u_hbm, out_pad_hbm,
                  idx_chunk_v, pairs_v, sorted_v, bidx_v, colbuf_v,
                  rows_seq_v, urows_v, hist_s, cum_s, csem, gsem, wsem):
    wid = lax.axis_index("s") * 2 + lax.axis_index("c")
    lo = wid * _CPW
    nc = jnp.minimum(lo + _CPW, _NCOLS) - lo
    iota = lax.iota(jnp.int32, _LANES)

    cnt = _scan_sort(item_hbm, idx_chunk_v, pairs_v, sorted_v, bidx_v,
                     hist_s, cum_s, lo, nc)

    def nonempty(cc):
        return cum_s[cc] < cum_s[cc + 1]

    def dot_group(grp):
        pltpu.async_copy(
            rows_u_hbm.at[bidx_v.at[grp]], urows_v, gsem).wait()
        for sub in range(128 // _LANES):
            rvec = sub * _LANES + iota
            acc = jnp.zeros((_LANES,), jnp.float32)
            for d in range(_DIM):
                dvec = jnp.full((_LANES,), d, jnp.int32)
                uu = plsc.load_gather(urows_v, [rvec, dvec])
                vv = plsc.load_gather(rows_seq_v, [rvec, dvec])
                acc = acc + uu * vv
            plsc.store_scatter(urows_v, [rvec, jnp.zeros((_LANES,), jnp.int32)],
                               acc)
        pltpu.async_copy(urows_v, out_pad_hbm.at[bidx_v.at[grp]], wsem).wait()

    for p in range(4):
        @pl.when((p < nc) & nonempty(p))
        def _():
            _col_dma(itabT_hbm, lo, p, colbuf_v, p, csem)

    def col_body(cc, _):
        begin = cum_s[cc]
        end = cum_s[cc + 1]
        slot = lax.rem(cc, 5)

        @pl.when((cc + 4 < nc) & nonempty(cc + 4))
        def _prefetch():
            _col_dma(itabT_hbm, lo, cc + 4, colbuf_v, lax.rem(cc + 4, 5),
                     csem)

        @pl.when(begin < end)
        def _process():
            pltpu.make_async_copy(
                itabT_hbm.at[:, pl.ds(lo * 128, 128)],
                colbuf_v.at[slot], csem).wait()

            def pair_body(i, _):
                v = sorted_v[pl.ds((i >> 4) << 4, _LANES)]
                lb = _select_scalar(v, i & 15)
                lane = lb & 127
                r = i & 127
                _extract_row(colbuf_v.at[slot], lane, rows_seq_v, r)

                @pl.when((i & 127) == 127)
                def _flush():
                    dot_group(i >> 7)
                return 0

            lax.fori_loop(begin, end, pair_body, 0)
        return 0

    lax.fori_loop(0, nc, col_body, 0)

    @pl.when((cnt & 127) != 0)
    def _tail():
        dot_group(cnt >> 7)


_COMPILER_PARAMS = pltpu.CompilerParams(
    use_tc_tiling_on_sc=True,
    needs_layout_passes=False,
)


@functools.partial(jax.jit, static_argnames=())
def kernel(user, item, user_table, item_table):
    mesh = plsc.VectorSubcoreMesh(core_axis_name="c", subcore_axis_name="s")

    run_users = pl.kernel(
        _users_kernel,
        mesh=mesh,
        compiler_params=_COMPILER_PARAMS,
        out_type=jax.ShapeDtypeStruct((_BATCH + 1, 128), jnp.float32),
        scratch_types=[
            pltpu.VMEM((_ICHUNK,), jnp.int32),
            pltpu.VMEM((_PAIR_CAP,), jnp.int32),
            pltpu.VMEM((_PAIR_CAP,), jnp.int32),
            pltpu.VMEM((_NGRP + 1, 128), jnp.int32),
            pltpu.VMEM((5, _DIM, 128), jnp.float32),
            pltpu.VMEM((128, 128), jnp.float32),
            pltpu.SMEM((256,), jnp.int32),
            pltpu.SMEM((256,), jnp.int32),
            pltpu.SemaphoreType.DMA,
            pltpu.SemaphoreType.DMA,
        ],
    )
    rows_u = run_users(user, user_table.T)

    run_items = pl.kernel(
        _items_kernel,
        mesh=mesh,
        compiler_params=_COMPILER_PARAMS,
        out_type=jax.ShapeDtypeStruct((_BATCH + 1, 128), jnp.float32),
        scratch_types=[
            pltpu.VMEM((_ICHUNK,), jnp.int32),
            pltpu.VMEM((_PAIR_CAP,), jnp.int32),
            pltpu.VMEM((_PAIR_CAP,), jnp.int32),
            pltpu.VMEM((_NGRP + 1, 128), jnp.int32),
            pltpu.VMEM((5, _DIM, 128), jnp.float32),
            pltpu.VMEM((128, 128), jnp.float32),
            pltpu.VMEM((128, 128), jnp.float32),
            pltpu.SMEM((256,), jnp.int32),
            pltpu.SMEM((256,), jnp.int32),
            pltpu.SemaphoreType.DMA,
            pltpu.SemaphoreType.DMA,
            pltpu.SemaphoreType.DMA,
        ],
    )
    out_pad = run_items(item, item_table.T, rows_u)
    return out_pad[:_BATCH, 0]
